# baseline (speedup 1.0000x reference)
import functools

import jax
import jax.numpy as jnp
from jax import lax
from jax.experimental import pallas as pl
from jax.experimental.pallas import tpu as pltpu
from jax.experimental.pallas import tpu_sc as plsc

_L = 200
_PAD = 256
_DIM = 64
_BLK = 2048
_VOCAB = 100000
_NSTEP = -(-_VOCAB // _BLK)
_VP = _NSTEP * _BLK


def _wbuild_body(idx_hbm, wts_hbm, zeros_hbm, w2_hbm, acc_v, idx_v, wts_v):
    cid = lax.axis_index("c")
    sid = lax.axis_index("s")

    @pl.when(sid == 0)
    def _():
        pltpu.sync_copy(zeros_hbm, acc_v)
        pltpu.sync_copy(idx_hbm, idx_v)
        pltpu.sync_copy(wts_hbm, wts_v)
        base = cid * _PAD
        for k in range(_PAD // 16):
            iv = idx_v[pl.ds(base + k * 16, 16)]
            wv = wts_v[pl.ds(base + k * 16, 16)]
            plsc.addupdate_scatter(acc_v, [iv], wv)
        pltpu.sync_copy(acc_v, w2_hbm.at[cid])


_wbuild = pl.kernel(
    _wbuild_body,
    out_type=jax.ShapeDtypeStruct((2, _VP), jnp.float32),
    scratch_types=[
        pltpu.VMEM((_VP,), jnp.float32),
        pltpu.VMEM((2 * _PAD,), jnp.int32),
        pltpu.VMEM((2 * _PAD,), jnp.float32),
    ],
    mesh=plsc.VectorSubcoreMesh(core_axis_name="c", subcore_axis_name="s"),
    compiler_params=pltpu.CompilerParams(needs_layout_passes=False),
)


_TAIL = _VOCAB - (_NSTEP - 1) * _BLK


def _wsum_body(x_ref, w_ref, o_ref, acc):
    pid = pl.program_id(0)

    @pl.when(pid == 0)
    def _():
        acc[...] = jnp.zeros_like(acc)

    @pl.when(pid == _NSTEP - 1)
    def _():
        x_ref[:, pl.ds(_TAIL, _BLK - _TAIL)] = jnp.zeros(
            (_DIM, _BLK - _TAIL), jnp.float32)

    acc[...] += jax.lax.dot_general(
        w_ref[...], x_ref[...], (((1,), (1,)), ((), ())),
        preferred_element_type=jnp.float32)

    @pl.when(pid == _NSTEP - 1)
    def _():
        o_ref[0, pl.ds(0, _DIM)] = acc[0, :]
        o_ref[0, pl.ds(_DIM, _DIM)] = acc[1, :]


def _wsum(table_t, w2):
    return pl.pallas_call(
        _wsum_body,
        grid=(_NSTEP,),
        in_specs=[
            pl.BlockSpec((_DIM, _BLK), lambda i: (0, i)),
            pl.BlockSpec((2, _BLK), lambda i: (0, i)),
        ],
        out_specs=pl.BlockSpec((1, 2 * _DIM), lambda i: (0, 0)),
        out_shape=jax.ShapeDtypeStruct((1, 2 * _DIM), jnp.float32),
        scratch_shapes=[
            pltpu.VMEM((2, _DIM), jnp.float32),
        ],
    )(table_t, w2)


def kernel(s1, s2, table):
    pad = jnp.zeros((_PAD - _L,), jnp.int32)
    idx = jnp.concatenate([s1.astype(jnp.int32), pad,
                           s2.astype(jnp.int32), pad])
    pos = jnp.arange(_PAD, dtype=jnp.int32)
    wts1 = jnp.where(pos < _L, jnp.float32(1.0 / _L), jnp.float32(0.0))
    wts = jnp.concatenate([wts1, wts1])
    zeros = jnp.zeros((_VP,), jnp.float32)
    w2 = _wbuild(idx, wts, zeros)
    table_t = jnp.transpose(table)
    return _wsum(table_t, w2).reshape(2 * _DIM)


# speedup vs baseline: 1.4266x; 1.4266x over previous
import functools

import jax
import jax.numpy as jnp
from jax import lax
from jax.experimental import pallas as pl
from jax.experimental.pallas import tpu as pltpu
from jax.experimental.pallas import tpu_sc as plsc

_L = 200
_PAD = 256
_DIM = 64
_BLK = 8192
_VOCAB = 100000
_NSTEP = -(-_VOCAB // _BLK)
_VP = _NSTEP * _BLK


def _wbuild_body(idx_hbm, wts_hbm, zeros_hbm, w2_hbm, acc_v, idx_v, wts_v):
    cid = lax.axis_index("c")
    sid = lax.axis_index("s")

    @pl.when(sid == 0)
    def _():
        pltpu.sync_copy(zeros_hbm, acc_v)
        pltpu.sync_copy(idx_hbm, idx_v)
        pltpu.sync_copy(wts_hbm, wts_v)
        base = cid * _PAD
        for k in range(_PAD // 16):
            iv = idx_v[pl.ds(base + k * 16, 16)]
            wv = wts_v[pl.ds(base + k * 16, 16)]
            plsc.addupdate_scatter(acc_v, [iv], wv)
        pltpu.sync_copy(acc_v, w2_hbm.at[cid])


_wbuild = pl.kernel(
    _wbuild_body,
    out_type=jax.ShapeDtypeStruct((2, _VP), jnp.float32),
    scratch_types=[
        pltpu.VMEM((_VP,), jnp.float32),
        pltpu.VMEM((2 * _PAD,), jnp.int32),
        pltpu.VMEM((2 * _PAD,), jnp.float32),
    ],
    mesh=plsc.VectorSubcoreMesh(core_axis_name="c", subcore_axis_name="s"),
    compiler_params=pltpu.CompilerParams(needs_layout_passes=False),
)


_TAIL = _VOCAB - (_NSTEP - 1) * _BLK


def _wsum_body(x_ref, w_ref, o_ref, acc):
    pid = pl.program_id(0)

    @pl.when(pid == 0)
    def _():
        acc[...] = jnp.zeros_like(acc)

    @pl.when(pid == _NSTEP - 1)
    def _():
        x_ref[:, pl.ds(_TAIL, _BLK - _TAIL)] = jnp.zeros(
            (_DIM, _BLK - _TAIL), jnp.float32)

    acc[...] += jax.lax.dot_general(
        w_ref[...], x_ref[...], (((1,), (1,)), ((), ())),
        preferred_element_type=jnp.float32)

    @pl.when(pid == _NSTEP - 1)
    def _():
        o_ref[0, pl.ds(0, _DIM)] = acc[0, :]
        o_ref[0, pl.ds(_DIM, _DIM)] = acc[1, :]


def _wsum(table_t, w2):
    return pl.pallas_call(
        _wsum_body,
        grid=(_NSTEP,),
        in_specs=[
            pl.BlockSpec((_DIM, _BLK), lambda i: (0, i)),
            pl.BlockSpec((2, _BLK), lambda i: (0, i)),
        ],
        out_specs=pl.BlockSpec((1, 2 * _DIM), lambda i: (0, 0)),
        out_shape=jax.ShapeDtypeStruct((1, 2 * _DIM), jnp.float32),
        scratch_shapes=[
            pltpu.VMEM((2, _DIM), jnp.float32),
        ],
    )(table_t, w2)


def kernel(s1, s2, table):
    pad = jnp.zeros((_PAD - _L,), jnp.int32)
    idx = jnp.concatenate([s1.astype(jnp.int32), pad,
                           s2.astype(jnp.int32), pad])
    pos = jnp.arange(_PAD, dtype=jnp.int32)
    wts1 = jnp.where(pos < _L, jnp.float32(1.0 / _L), jnp.float32(0.0))
    wts = jnp.concatenate([wts1, wts1])
    zeros = jnp.zeros((_VP,), jnp.float32)
    w2 = _wbuild(idx, wts, zeros)
    table_t = jnp.transpose(table)
    return _wsum(table_t, w2).reshape(2 * _DIM)
